# baseline (speedup 1.0000x reference)
import functools

import jax
import jax.numpy as jnp
from jax.experimental import pallas as pl

H, KVH, DH = 16, 8, 128
EPS = 1e-6
NEG = -1e30

BM_PROJ = 256
BM_Q = 256
BK = 256


def _qkv_proj_kernel(x_ref, wq_ref, wk_ref, wv_ref, q_ref, k_ref, v_ref):
    xb = x_ref[...].astype(jnp.bfloat16)
    dims = (((1,), (1,)), ((), ()))
    q = jax.lax.dot_general(xb, wq_ref[...], dims,
                            preferred_element_type=jnp.float32)
    k = jax.lax.dot_general(xb, wk_ref[...], dims,
                            preferred_element_type=jnp.float32)
    v = jax.lax.dot_general(xb, wv_ref[...], dims,
                            preferred_element_type=jnp.float32)
    q_ref[...] = q.astype(jnp.bfloat16)
    k_ref[...] = k.astype(jnp.bfloat16)
    v_ref[...] = v.astype(jnp.bfloat16)


def _attn_kernel(q_ref, k_ref, v_ref, qw_ref, kw_ref, o_ref):
    i = pl.program_id(1)
    scale = DH ** -0.5

    qb = q_ref[...].astype(jnp.float32)
    qvar = jnp.mean(qb * qb, axis=-1, keepdims=True)
    qn = qb * jax.lax.rsqrt(qvar + EPS) * qw_ref[...]
    qn = (qn * scale).astype(jnp.bfloat16)

    kw = kw_ref[...]
    row_ids = i * BM_Q + jax.lax.broadcasted_iota(jnp.int32, (BM_Q, BK), 0)
    col_iota = jax.lax.broadcasted_iota(jnp.int32, (BM_Q, BK), 1)

    def body(j, carry):
        m, l, acc = carry
        kc = k_ref[pl.ds(j * BK, BK), :].astype(jnp.float32)
        kvar = jnp.mean(kc * kc, axis=-1, keepdims=True)
        kn = (kc * jax.lax.rsqrt(kvar + EPS) * kw).astype(jnp.bfloat16)
        s = jax.lax.dot_general(qn, kn, (((1,), (1,)), ((), ())),
                                preferred_element_type=jnp.float32)
        s = jnp.where(row_ids >= j * BK + col_iota, s, NEG)
        m2 = jnp.maximum(m, jnp.max(s, axis=-1, keepdims=True))
        p = jnp.exp(s - m2)
        alpha = jnp.exp(m - m2)
        l2 = l * alpha + jnp.sum(p, axis=-1, keepdims=True)
        vc = v_ref[pl.ds(j * BK, BK), :]
        acc2 = acc * alpha + jax.lax.dot_general(
            p.astype(jnp.bfloat16), vc, (((1,), (0,)), ((), ())),
            preferred_element_type=jnp.float32)
        return m2, l2, acc2

    m0 = jnp.full((BM_Q, 1), NEG, dtype=jnp.float32)
    l0 = jnp.zeros((BM_Q, 1), dtype=jnp.float32)
    a0 = jnp.zeros((BM_Q, DH), dtype=jnp.float32)
    n_chunks = (i + 1) * (BM_Q // BK)
    m, l, acc = jax.lax.fori_loop(0, n_chunks, body, (m0, l0, a0))
    o_ref[...] = (acc / l).astype(jnp.bfloat16)


def _out_proj_kernel(a_ref, wo_ref, o_ref):
    o_ref[...] = jax.lax.dot_general(
        a_ref[...], wo_ref[...], (((1,), (1,)), ((), ())),
        preferred_element_type=jnp.float32)


@functools.partial(jax.jit, static_argnums=())
def kernel(x, Wq, Wk, Wv, Wo, q_norm_w, k_norm_w):
    b, s, d = x.shape
    x2 = x.reshape(s, d)
    wq = Wq.astype(jnp.bfloat16)
    wk = Wk.astype(jnp.bfloat16)
    wv = Wv.astype(jnp.bfloat16)
    wo = Wo.astype(jnp.bfloat16)
    qw = q_norm_w.reshape(1, DH)
    kw = k_norm_w.reshape(1, DH)

    n_row_blocks = s // BM_PROJ
    q, k, v = pl.pallas_call(
        _qkv_proj_kernel,
        grid=(n_row_blocks,),
        in_specs=[
            pl.BlockSpec((BM_PROJ, d), lambda i: (i, 0)),
            pl.BlockSpec((H * DH, d), lambda i: (0, 0)),
            pl.BlockSpec((KVH * DH, d), lambda i: (0, 0)),
            pl.BlockSpec((KVH * DH, d), lambda i: (0, 0)),
        ],
        out_specs=[
            pl.BlockSpec((BM_PROJ, H * DH), lambda i: (i, 0)),
            pl.BlockSpec((BM_PROJ, KVH * DH), lambda i: (i, 0)),
            pl.BlockSpec((BM_PROJ, KVH * DH), lambda i: (i, 0)),
        ],
        out_shape=[
            jax.ShapeDtypeStruct((s, H * DH), jnp.bfloat16),
            jax.ShapeDtypeStruct((s, KVH * DH), jnp.bfloat16),
            jax.ShapeDtypeStruct((s, KVH * DH), jnp.bfloat16),
        ],
    )(x2, wq, wk, wv)

    n_q_blocks = s // BM_Q
    groups = H // KVH
    attn = pl.pallas_call(
        _attn_kernel,
        grid=(H, n_q_blocks),
        in_specs=[
            pl.BlockSpec((BM_Q, DH), lambda h, i: (i, h)),
            pl.BlockSpec((s, DH), lambda h, i: (0, h // groups)),
            pl.BlockSpec((s, DH), lambda h, i: (0, h // groups)),
            pl.BlockSpec((1, DH), lambda h, i: (0, 0)),
            pl.BlockSpec((1, DH), lambda h, i: (0, 0)),
        ],
        out_specs=pl.BlockSpec((BM_Q, DH), lambda h, i: (i, h)),
        out_shape=jax.ShapeDtypeStruct((s, H * DH), jnp.bfloat16),
    )(q, k, v, qw, kw)

    out = pl.pallas_call(
        _out_proj_kernel,
        grid=(n_row_blocks,),
        in_specs=[
            pl.BlockSpec((BM_PROJ, H * DH), lambda i: (i, 0)),
            pl.BlockSpec((d, H * DH), lambda i: (0, 0)),
        ],
        out_specs=pl.BlockSpec((BM_PROJ, d), lambda i: (i, 0)),
        out_shape=jax.ShapeDtypeStruct((s, d), jnp.float32),
    )(attn, wo)

    return out.reshape(b, s, d)


# speedup vs baseline: 2.1751x; 2.1751x over previous
import jax
import jax.numpy as jnp
from jax.experimental import pallas as pl

H, KVH, DH = 16, 8, 128
EPS = 1e-6
NEG = -1e30

BM_PROJ = 256
BM_Q = 512
BK = 512


def _rms_norm_heads(t, w, extra_scale):
    rows = t.shape[0]
    n = t.shape[1] // DH
    t3 = t.reshape(rows, n, DH)
    var = jnp.mean(t3 * t3, axis=-1, keepdims=True)
    t3 = t3 * (jax.lax.rsqrt(var + EPS) * extra_scale)
    return (t3 * w.reshape(1, 1, DH)).reshape(rows, n * DH)


def _qkv_proj_kernel(x_ref, wq_ref, wk_ref, wv_ref, qw_ref, kw_ref,
                     q_ref, k_ref, v_ref):
    xb = x_ref[...].astype(jnp.bfloat16)
    dims = (((1,), (1,)), ((), ()))
    q = jax.lax.dot_general(xb, wq_ref[...], dims,
                            preferred_element_type=jnp.float32)
    k = jax.lax.dot_general(xb, wk_ref[...], dims,
                            preferred_element_type=jnp.float32)
    v = jax.lax.dot_general(xb, wv_ref[...], dims,
                            preferred_element_type=jnp.float32)
    qn = _rms_norm_heads(q, qw_ref[...], DH ** -0.5)
    kn = _rms_norm_heads(k, kw_ref[...], 1.0)
    q_ref[...] = qn.astype(jnp.bfloat16)
    k_ref[...] = kn.astype(jnp.bfloat16)
    v_ref[...] = v.astype(jnp.bfloat16)


def _attn_kernel(q_ref, k_ref, v_ref, o_ref):
    i = pl.program_id(1)
    qb = q_ref[...]

    def body(j, carry):
        l, acc = carry
        kc = k_ref[pl.ds(j * BK, BK), :]
        s = jax.lax.dot_general(qb, kc, (((1,), (1,)), ((), ())),
                                preferred_element_type=jnp.float32)
        p = jnp.exp(s)
        l = l + jnp.sum(p, axis=-1, keepdims=True)
        vc = v_ref[pl.ds(j * BK, BK), :]
        acc = acc + jax.lax.dot_general(
            p.astype(jnp.bfloat16), vc, (((1,), (0,)), ((), ())),
            preferred_element_type=jnp.float32)
        return l, acc

    l0 = jnp.zeros((BM_Q, 1), dtype=jnp.float32)
    a0 = jnp.zeros((BM_Q, DH), dtype=jnp.float32)
    l, acc = jax.lax.fori_loop(0, i, body, (l0, a0))

    kc = k_ref[pl.ds(i * BK, BK), :]
    s = jax.lax.dot_general(qb, kc, (((1,), (1,)), ((), ())),
                            preferred_element_type=jnp.float32)
    row = jax.lax.broadcasted_iota(jnp.int32, (BM_Q, BK), 0)
    col = jax.lax.broadcasted_iota(jnp.int32, (BM_Q, BK), 1)
    s = jnp.where(row >= col, s, NEG)
    p = jnp.exp(s)
    l = l + jnp.sum(p, axis=-1, keepdims=True)
    vc = v_ref[pl.ds(i * BK, BK), :]
    acc = acc + jax.lax.dot_general(
        p.astype(jnp.bfloat16), vc, (((1,), (0,)), ((), ())),
        preferred_element_type=jnp.float32)

    o_ref[...] = (acc / l).astype(jnp.bfloat16)


def _out_proj_kernel(a_ref, wo_ref, o_ref):
    o_ref[...] = jax.lax.dot_general(
        a_ref[...], wo_ref[...], (((1,), (1,)), ((), ())),
        preferred_element_type=jnp.float32)


def kernel(x, Wq, Wk, Wv, Wo, q_norm_w, k_norm_w):
    b, s, d = x.shape
    x2 = x.reshape(s, d)
    wq = Wq.astype(jnp.bfloat16)
    wk = Wk.astype(jnp.bfloat16)
    wv = Wv.astype(jnp.bfloat16)
    wo = Wo.astype(jnp.bfloat16)
    qw = q_norm_w.reshape(1, DH)
    kw = k_norm_w.reshape(1, DH)

    n_row_blocks = s // BM_PROJ
    q, k, v = pl.pallas_call(
        _qkv_proj_kernel,
        grid=(n_row_blocks,),
        in_specs=[
            pl.BlockSpec((BM_PROJ, d), lambda i: (i, 0)),
            pl.BlockSpec((H * DH, d), lambda i: (0, 0)),
            pl.BlockSpec((KVH * DH, d), lambda i: (0, 0)),
            pl.BlockSpec((KVH * DH, d), lambda i: (0, 0)),
            pl.BlockSpec((1, DH), lambda i: (0, 0)),
            pl.BlockSpec((1, DH), lambda i: (0, 0)),
        ],
        out_specs=[
            pl.BlockSpec((BM_PROJ, H * DH), lambda i: (i, 0)),
            pl.BlockSpec((BM_PROJ, KVH * DH), lambda i: (i, 0)),
            pl.BlockSpec((BM_PROJ, KVH * DH), lambda i: (i, 0)),
        ],
        out_shape=[
            jax.ShapeDtypeStruct((s, H * DH), jnp.bfloat16),
            jax.ShapeDtypeStruct((s, KVH * DH), jnp.bfloat16),
            jax.ShapeDtypeStruct((s, KVH * DH), jnp.bfloat16),
        ],
    )(x2, wq, wk, wv, qw, kw)

    n_q_blocks = s // BM_Q
    groups = H // KVH
    attn = pl.pallas_call(
        _attn_kernel,
        grid=(H, n_q_blocks),
        in_specs=[
            pl.BlockSpec((BM_Q, DH), lambda h, i: (i, h)),
            pl.BlockSpec((s, DH), lambda h, i: (0, h // groups)),
            pl.BlockSpec((s, DH), lambda h, i: (0, h // groups)),
        ],
        out_specs=pl.BlockSpec((BM_Q, DH), lambda h, i: (i, h)),
        out_shape=jax.ShapeDtypeStruct((s, H * DH), jnp.bfloat16),
    )(q, k, v)

    out = pl.pallas_call(
        _out_proj_kernel,
        grid=(n_row_blocks,),
        in_specs=[
            pl.BlockSpec((BM_PROJ, H * DH), lambda i: (i, 0)),
            pl.BlockSpec((d, H * DH), lambda i: (0, 0)),
        ],
        out_specs=pl.BlockSpec((BM_PROJ, d), lambda i: (i, 0)),
        out_shape=jax.ShapeDtypeStruct((s, d), jnp.float32),
    )(attn, wo)

    return out.reshape(b, s, d)
